# baseline (speedup 1.0000x reference)
import functools

import jax
import jax.numpy as jnp
from jax import lax
from jax.experimental import pallas as pl
from jax.experimental.pallas import tpu as pltpu
from jax.experimental.pallas import tpu_sc as plsc

_N = 10000
_E = 320000
_D = 128

_NC = 2
_NS = 16
_NW = _NC * _NS
_CHUNK = 80
_NCHUNK = 128
_PAIRS = _NCHUNK // 2
_EPW = _NCHUNK * _CHUNK
_EPAD = _NW * _EPW
_NPAD = 10240
_RPS = _NPAD // _NS


def _sc_aggregate(x, src, dst):
    mesh = plsc.VectorSubcoreMesh(core_axis_name="c", subcore_axis_name="s")

    @functools.partial(
        pl.kernel,
        mesh=mesh,
        out_type=jax.ShapeDtypeStruct((2, _NPAD, _D), jnp.float32),
        scratch_types=[
            pltpu.VMEM((_CHUNK,), jnp.int32),
            pltpu.VMEM((_CHUNK,), jnp.int32),
            pltpu.VMEM((_CHUNK,), jnp.int32),
            pltpu.VMEM((_CHUNK,), jnp.int32),
            pltpu.VMEM((_CHUNK, _D), jnp.float32),
            pltpu.VMEM((_CHUNK, _D), jnp.float32),
            pltpu.VMEM_SHARED((_NPAD, _D), jnp.float32),
            pltpu.SemaphoreType.DMA,
            pltpu.SemaphoreType.DMA,
            pltpu.SemaphoreType.DMA,
            pltpu.SemaphoreType.DMA,
        ],
    )
    def agg(x_hbm, src_hbm, dst_hbm, out_hbm, srcA, srcB, dstA, dstB,
            rows0, rows1, acc_s, semg0, semg1, semiA, semiB):
        c = lax.axis_index("c")
        s = lax.axis_index("s")
        wid = c * _NS + s
        ebase = wid * _EPW

        def start_idx(j, sbuf, dbuf, sem):
            pltpu.async_copy(src_hbm.at[pl.ds(ebase + j * _CHUNK, _CHUNK)], sbuf, sem)
            pltpu.async_copy(dst_hbm.at[pl.ds(ebase + j * _CHUNK, _CHUNK)], dbuf, sem)

        def wait_idx(sbuf, dbuf, sem):
            pltpu.make_async_copy(src_hbm.at[pl.ds(0, _CHUNK)], sbuf, sem).wait()
            pltpu.make_async_copy(src_hbm.at[pl.ds(0, _CHUNK)], dbuf, sem).wait()

        def start_g(sbuf, buf, sem):
            pltpu.async_copy(x_hbm.at[sbuf], buf, sem)

        def wait_g(buf, sem):
            pltpu.make_async_copy(x_hbm.at[pl.ds(0, _CHUNK)], buf, sem).wait()

        def scat(dbuf, buf):
            pltpu.sync_copy(buf, acc_s.at[dbuf], add=True)

        start_idx(0, srcA, dstA, semiA)

        def zstore(i, carry):
            rows0[i // (_D // 16), pl.ds((i % (_D // 16)) * 16, 16)] = jnp.zeros(
                (16,), jnp.float32)
            return carry

        lax.fori_loop(0, _CHUNK * (_D // 16), zstore, None)

        def zcopy(j, carry):
            pltpu.sync_copy(rows0, acc_s.at[pl.ds(s * _RPS + j * _CHUNK, _CHUNK)])
            return carry

        lax.fori_loop(0, _RPS // _CHUNK, zcopy, None)
        plsc.subcore_barrier()

        wait_idx(srcA, dstA, semiA)
        start_g(srcA, rows0, semg0)
        start_idx(1, srcB, dstB, semiB)

        def pair(i, carry):
            j0 = 2 * i
            wait_idx(srcB, dstB, semiB)
            start_g(srcB, rows1, semg1)
            wait_g(rows0, semg0)
            scat(dstA, rows0)

            @pl.when(i < _PAIRS - 1)
            def _():
                start_idx(j0 + 2, srcA, dstA, semiA)

            wait_g(rows1, semg1)
            scat(dstB, rows1)

            @pl.when(i < _PAIRS - 1)
            def _():
                start_idx(j0 + 3, srcB, dstB, semiB)
                wait_idx(srcA, dstA, semiA)
                start_g(srcA, rows0, semg0)

            return carry

        lax.fori_loop(0, _PAIRS, pair, None)
        plsc.subcore_barrier()

        pltpu.sync_copy(
            acc_s.at[pl.ds(s * _RPS, _RPS)],
            out_hbm.at[c, pl.ds(s * _RPS, _RPS)],
        )

    return agg(x, src, dst)


def _tc_combine(partials, W, b2):
    bn = 1000
    grid = (_N // bn,)

    def body(p0_ref, p1_ref, w_ref, b_ref, o_ref):
        a = p0_ref[0] + p1_ref[0]
        h = lax.dot_general(a, w_ref[...], (((1,), (1,)), ((), ())),
                            preferred_element_type=jnp.float32)
        o_ref[...] = h + b_ref[...]

    return pl.pallas_call(
        body,
        grid=grid,
        in_specs=[
            pl.BlockSpec((1, bn, _D), lambda i: (0, i, 0)),
            pl.BlockSpec((1, bn, _D), lambda i: (1, i, 0)),
            pl.BlockSpec((_D, _D), lambda i: (0, 0)),
            pl.BlockSpec((1, _D), lambda i: (0, 0)),
        ],
        out_specs=pl.BlockSpec((bn, _D), lambda i: (i, 0)),
        out_shape=jax.ShapeDtypeStruct((_N, _D), jnp.float32),
    )(partials, partials, W, b2)


@jax.jit
def kernel(x, edge_index, W, b):
    src = edge_index[0]
    dst = edge_index[1]
    pad = _EPAD - _E
    pad_dst = _N + (jnp.arange(pad, dtype=jnp.int32) % (_NPAD - _N))
    srcp = jnp.concatenate([src, jnp.zeros((pad,), jnp.int32)])
    dstp = jnp.concatenate([dst, pad_dst])
    partials = _sc_aggregate(x, srcp, dstp)
    out = _tc_combine(partials, W, b.reshape(1, _D))
    return (out,)


# speedup vs baseline: 3.3722x; 3.3722x over previous
import functools

import jax
import jax.numpy as jnp
from jax import lax
from jax.experimental import pallas as pl
from jax.experimental.pallas import tpu as pltpu
from jax.experimental.pallas import tpu_sc as plsc

_N = 10000
_E = 320000
_D = 128

_NC = 2
_NS = 16
_NW = _NC * _NS
_EPW = _E // _NW
_CHUNK = 80
_NCHUNK = _EPW // _CHUNK
_QUADS = (_NCHUNK - 5) // 4
_NPAD = 10240
_RPS = _NPAD // _NS


def _sc_aggregate(x, src, dst):
    mesh = plsc.VectorSubcoreMesh(core_axis_name="c", subcore_axis_name="s")

    @functools.partial(
        pl.kernel,
        mesh=mesh,
        out_type=jax.ShapeDtypeStruct((2, _NPAD, _D), jnp.float32),
        scratch_types=[
            pltpu.VMEM((_CHUNK,), jnp.int32),
            pltpu.VMEM((_CHUNK,), jnp.int32),
            pltpu.VMEM((_CHUNK,), jnp.int32),
            pltpu.VMEM((_CHUNK,), jnp.int32),
            pltpu.VMEM((_CHUNK,), jnp.int32),
            pltpu.VMEM((_CHUNK,), jnp.int32),
            pltpu.VMEM((_CHUNK,), jnp.int32),
            pltpu.VMEM((_CHUNK,), jnp.int32),
            pltpu.VMEM((_CHUNK, _D), jnp.float32),
            pltpu.VMEM((_CHUNK, _D), jnp.float32),
            pltpu.VMEM_SHARED((_NPAD, _D), jnp.float32),
            pltpu.SemaphoreType.DMA,
            pltpu.SemaphoreType.DMA,
            pltpu.SemaphoreType.DMA,
            pltpu.SemaphoreType.DMA,
            pltpu.SemaphoreType.DMA,
            pltpu.SemaphoreType.DMA,
        ],
    )
    def agg(x_hbm, src_hbm, dst_hbm, out_hbm, srcA, srcB, srcC, srcD,
            dstA, dstB, dstC, dstD, rows0, rows1, acc_s,
            semg0, semg1, semiA, semiB, semiC, semiD):
        c = lax.axis_index("c")
        s = lax.axis_index("s")
        wid = c * _NS + s
        ebase = wid * _EPW

        def start_idx(j, sbuf, dbuf, sem):
            pltpu.async_copy(src_hbm.at[pl.ds(ebase + j * _CHUNK, _CHUNK)], sbuf, sem)
            pltpu.async_copy(dst_hbm.at[pl.ds(ebase + j * _CHUNK, _CHUNK)], dbuf, sem)

        def wait_idx(sbuf, dbuf, sem):
            pltpu.make_async_copy(src_hbm.at[pl.ds(0, _CHUNK)], sbuf, sem).wait()
            pltpu.make_async_copy(src_hbm.at[pl.ds(0, _CHUNK)], dbuf, sem).wait()

        def start_g(sbuf, buf, sem):
            pltpu.async_copy(x_hbm.at[sbuf], buf, sem)

        def wait_g(buf, sem):
            pltpu.make_async_copy(x_hbm.at[pl.ds(0, _CHUNK)], buf, sem).wait()

        def scat(dbuf, buf):
            pltpu.sync_copy(buf, acc_s.at[dbuf], add=True)

        sets = ((srcA, dstA, semiA), (srcB, dstB, semiB),
                (srcC, dstC, semiC), (srcD, dstD, semiD))

        for k in range(4):
            start_idx(k, *sets[k])

        def zstore(i, carry):
            rows0[i // (_D // 16), pl.ds((i % (_D // 16)) * 16, 16)] = jnp.zeros(
                (16,), jnp.float32)
            return carry

        lax.fori_loop(0, _CHUNK * (_D // 16), zstore, None)

        def zcopy(j, carry):
            pltpu.sync_copy(rows0, acc_s.at[pl.ds(s * _RPS + j * _CHUNK, _CHUNK)])
            return carry

        lax.fori_loop(0, _RPS // _CHUNK, zcopy, None)

        wait_idx(*sets[0])
        start_g(srcA, rows0, semg0)
        plsc.subcore_barrier()

        rows = (rows0, rows1)
        semg = (semg0, semg1)

        def quad(i, carry):
            q0 = 4 * i
            for k in range(4):
                s_cur, d_cur, _ = sets[k]
                s_nxt, d_nxt, sem_nxt = sets[(k + 1) % 4]
                r_cur, r_nxt = rows[k % 2], rows[(k + 1) % 2]
                if k < 3:
                    wait_idx(*sets[k + 1])
                    start_g(s_nxt, r_nxt, semg[(k + 1) % 2])
                wait_g(r_cur, semg[k % 2])
                scat(d_cur, r_cur)
                start_idx(q0 + k + 4, s_cur, d_cur, sets[k][2])
            wait_idx(*sets[0])
            start_g(srcA, rows0, semg0)
            return carry

        lax.fori_loop(0, _QUADS, quad, None)

        for t in range(5):
            s_cur, d_cur, _ = sets[t % 4]
            r_cur = rows[t % 2]
            if 0 < t < 4:
                wait_idx(*sets[(t + 1) % 4])
                start_g(sets[(t + 1) % 4][0], rows[(t + 1) % 2], semg[(t + 1) % 2])
            elif t == 0:
                wait_idx(*sets[1])
                start_g(srcB, rows1, semg1)
            wait_g(r_cur, semg[t % 2])
            scat(d_cur, r_cur)
            if t == 0:
                start_idx(_NCHUNK - 1, *sets[0])
        plsc.subcore_barrier()

        pltpu.sync_copy(
            acc_s.at[pl.ds(s * _RPS, _RPS)],
            out_hbm.at[c, pl.ds(s * _RPS, _RPS)],
        )

    return agg(x, src, dst)


def _tc_combine(partials, W, b2):
    bn = 1000
    grid = (_N // bn,)

    def body(p0_ref, p1_ref, w_ref, b_ref, o_ref):
        a = p0_ref[0] + p1_ref[0]
        h = lax.dot_general(a, w_ref[...], (((1,), (1,)), ((), ())),
                            preferred_element_type=jnp.float32)
        o_ref[...] = h + b_ref[...]

    return pl.pallas_call(
        body,
        grid=grid,
        in_specs=[
            pl.BlockSpec((1, bn, _D), lambda i: (0, i, 0)),
            pl.BlockSpec((1, bn, _D), lambda i: (1, i, 0)),
            pl.BlockSpec((_D, _D), lambda i: (0, 0)),
            pl.BlockSpec((1, _D), lambda i: (0, 0)),
        ],
        out_specs=pl.BlockSpec((bn, _D), lambda i: (i, 0)),
        out_shape=jax.ShapeDtypeStruct((_N, _D), jnp.float32),
    )(partials, partials, W, b2)


@jax.jit
def kernel(x, edge_index, W, b):
    src = edge_index[0]
    dst = edge_index[1]
    partials = _sc_aggregate(x, src, dst)
    out = _tc_combine(partials, W, b.reshape(1, _D))
    return (out,)


# speedup vs baseline: 3.7499x; 1.1120x over previous
import functools

import jax
import jax.numpy as jnp
from jax import lax
from jax.experimental import pallas as pl
from jax.experimental.pallas import tpu as pltpu
from jax.experimental.pallas import tpu_sc as plsc

_N = 10000
_E = 320000
_D = 128

_NC = 2
_NS = 16
_NW = _NC * _NS
_EPW = _E // _NW
_CHUNK = 80
_NCHUNK = _EPW // _CHUNK
_QUADS = (_NCHUNK - 5) // 4
_NPAD = 10240
_RPS = _NPAD // _NS


def _sc_aggregate(x, src, dst):
    mesh = plsc.VectorSubcoreMesh(core_axis_name="c", subcore_axis_name="s")

    @functools.partial(
        pl.kernel,
        mesh=mesh,
        out_type=jax.ShapeDtypeStruct((2, _NPAD, _D), jnp.float32),
        scratch_types=[
            pltpu.VMEM((_CHUNK,), jnp.int32),
            pltpu.VMEM((_CHUNK,), jnp.int32),
            pltpu.VMEM((_CHUNK,), jnp.int32),
            pltpu.VMEM((_CHUNK,), jnp.int32),
            pltpu.VMEM((_CHUNK,), jnp.int32),
            pltpu.VMEM((_CHUNK,), jnp.int32),
            pltpu.VMEM((_CHUNK,), jnp.int32),
            pltpu.VMEM((_CHUNK,), jnp.int32),
            pltpu.VMEM((_CHUNK, _D), jnp.float32),
            pltpu.VMEM((_CHUNK, _D), jnp.float32),
            pltpu.VMEM((_CHUNK, _D), jnp.float32),
            pltpu.VMEM((_CHUNK, _D), jnp.float32),
            pltpu.VMEM_SHARED((_NPAD, _D), jnp.float32),
            pltpu.SemaphoreType.DMA,
            pltpu.SemaphoreType.DMA,
            pltpu.SemaphoreType.DMA,
            pltpu.SemaphoreType.DMA,
            pltpu.SemaphoreType.DMA,
            pltpu.SemaphoreType.DMA,
            pltpu.SemaphoreType.DMA,
            pltpu.SemaphoreType.DMA,
            pltpu.SemaphoreType.DMA,
            pltpu.SemaphoreType.DMA,
            pltpu.SemaphoreType.DMA,
            pltpu.SemaphoreType.DMA,
        ],
    )
    def agg(x_hbm, src_hbm, dst_hbm, out_hbm, srcA, srcB, srcC, srcD,
            dstA, dstB, dstC, dstD, rows0, rows1, rows2, rows3, acc_s,
            semg0, semg1, semg2, semg3, semsc0, semsc1, semsc2, semsc3,
            semiA, semiB, semiC, semiD):
        c = lax.axis_index("c")
        s = lax.axis_index("s")
        wid = c * _NS + s
        ebase = wid * _EPW

        def start_idx(j, sbuf, dbuf, sem):
            pltpu.async_copy(src_hbm.at[pl.ds(ebase + j * _CHUNK, _CHUNK)], sbuf, sem)
            pltpu.async_copy(dst_hbm.at[pl.ds(ebase + j * _CHUNK, _CHUNK)], dbuf, sem)

        def wait_idx(sbuf, dbuf, sem):
            pltpu.make_async_copy(src_hbm.at[pl.ds(0, _CHUNK)], sbuf, sem).wait()
            pltpu.make_async_copy(src_hbm.at[pl.ds(0, _CHUNK)], dbuf, sem).wait()

        def start_g(sbuf, buf, sem):
            pltpu.async_copy(x_hbm.at[sbuf], buf, sem)

        def wait_g(buf, sem):
            pltpu.make_async_copy(x_hbm.at[pl.ds(0, _CHUNK)], buf, sem).wait()

        def start_scat(dbuf, buf, sem):
            pltpu.async_copy(buf, acc_s.at[dbuf], sem, add=True)

        def wait_scat(buf, sem):
            pltpu.make_async_copy(buf, acc_s.at[pl.ds(0, _CHUNK)], sem).wait()

        sets = ((srcA, dstA, semiA), (srcB, dstB, semiB),
                (srcC, dstC, semiC), (srcD, dstD, semiD))
        rows = (rows0, rows1, rows2, rows3)
        semg = (semg0, semg1, semg2, semg3)
        semsc = (semsc0, semsc1, semsc2, semsc3)

        for k in range(4):
            start_idx(k, *sets[k])

        def zstore(i, carry):
            rows0[i // (_D // 16), pl.ds((i % (_D // 16)) * 16, 16)] = jnp.zeros(
                (16,), jnp.float32)
            return carry

        lax.fori_loop(0, _CHUNK * (_D // 16), zstore, None)

        def zcopy(j, carry):
            pltpu.sync_copy(rows0, acc_s.at[pl.ds(s * _RPS + j * _CHUNK, _CHUNK)])
            return carry

        lax.fori_loop(0, _RPS // _CHUNK, zcopy, None)

        wait_idx(*sets[0])
        start_g(srcA, rows0, semg0)
        plsc.subcore_barrier()

        for j in (0, 1):
            p, pn = j % 4, (j + 1) % 4
            wait_idx(*sets[pn])
            start_g(sets[pn][0], rows[pn], semg[pn])
            wait_g(rows[p], semg[p])
            start_scat(sets[p][1], rows[p], semsc[p])

        def quad(i, carry):
            j0 = 4 * i + 2
            for k in range(4):
                p = (2 + k) % 4
                pn = (3 + k) % 4
                pm = k
                wait_idx(*sets[pn])
                start_g(sets[pn][0], rows[pn], semg[pn])
                wait_g(rows[p], semg[p])
                start_scat(sets[p][1], rows[p], semsc[p])
                wait_scat(rows[pm], semsc[pm])
                start_idx(j0 + k + 2, sets[pm][0], sets[pm][1], sets[pm][2])
            return carry

        lax.fori_loop(0, _QUADS, quad, None)

        j = 122
        p, pn, pm = j % 4, (j + 1) % 4, (j + 2) % 4
        wait_idx(*sets[pn])
        start_g(sets[pn][0], rows[pn], semg[pn])
        wait_g(rows[p], semg[p])
        start_scat(sets[p][1], rows[p], semsc[p])
        wait_scat(rows[pm], semsc[pm])
        start_idx(124, sets[pm][0], sets[pm][1], sets[pm][2])
        j = 123
        p, pn, pm = j % 4, (j + 1) % 4, (j + 2) % 4
        wait_idx(*sets[pn])
        start_g(sets[pn][0], rows[pn], semg[pn])
        wait_g(rows[p], semg[p])
        start_scat(sets[p][1], rows[p], semsc[p])
        wait_scat(rows[pm], semsc[pm])
        wait_g(rows[0], semg[0])
        start_scat(sets[0][1], rows[0], semsc[0])
        wait_scat(rows[2], semsc[2])
        wait_scat(rows[3], semsc[3])
        wait_scat(rows[0], semsc[0])
        plsc.subcore_barrier()

        pltpu.sync_copy(
            acc_s.at[pl.ds(s * _RPS, _RPS)],
            out_hbm.at[c, pl.ds(s * _RPS, _RPS)],
        )

    return agg(x, src, dst)


def _tc_combine(partials, W, b2):
    bn = 1000
    grid = (_N // bn,)

    def body(p0_ref, p1_ref, w_ref, b_ref, o_ref):
        a = p0_ref[0] + p1_ref[0]
        h = lax.dot_general(a, w_ref[...], (((1,), (1,)), ((), ())),
                            preferred_element_type=jnp.float32)
        o_ref[...] = h + b_ref[...]

    return pl.pallas_call(
        body,
        grid=grid,
        in_specs=[
            pl.BlockSpec((1, bn, _D), lambda i: (0, i, 0)),
            pl.BlockSpec((1, bn, _D), lambda i: (1, i, 0)),
            pl.BlockSpec((_D, _D), lambda i: (0, 0)),
            pl.BlockSpec((1, _D), lambda i: (0, 0)),
        ],
        out_specs=pl.BlockSpec((bn, _D), lambda i: (i, 0)),
        out_shape=jax.ShapeDtypeStruct((_N, _D), jnp.float32),
    )(partials, partials, W, b2)


@jax.jit
def kernel(x, edge_index, W, b):
    src = edge_index[0]
    dst = edge_index[1]
    partials = _sc_aggregate(x, src, dst)
    out = _tc_combine(partials, W, b.reshape(1, _D))
    return (out,)


# speedup vs baseline: 3.8040x; 1.0144x over previous
import functools

import jax
import jax.numpy as jnp
from jax import lax
from jax.experimental import pallas as pl
from jax.experimental.pallas import tpu as pltpu
from jax.experimental.pallas import tpu_sc as plsc

_N = 10000
_E = 320000
_D = 128

_NC = 2
_NS = 16
_NW = _NC * _NS
_EPW = _E // _NW
_CHUNK = 80
_NCHUNK = _EPW // _CHUNK
_QUADS = (_NCHUNK - 5) // 4
_NPAD = 10240
_RPS = _NPAD // _NS


def _sc_aggregate(x, src, dst):
    mesh = plsc.VectorSubcoreMesh(core_axis_name="c", subcore_axis_name="s")

    @functools.partial(
        pl.kernel,
        mesh=mesh,
        out_type=jax.ShapeDtypeStruct((2, _NPAD, _D), jnp.float32),
        scratch_types=[
            pltpu.VMEM((_CHUNK,), jnp.int32),
            pltpu.VMEM((_CHUNK,), jnp.int32),
            pltpu.VMEM((_CHUNK,), jnp.int32),
            pltpu.VMEM((_CHUNK,), jnp.int32),
            pltpu.VMEM((_CHUNK,), jnp.int32),
            pltpu.VMEM((_CHUNK,), jnp.int32),
            pltpu.VMEM((_CHUNK,), jnp.int32),
            pltpu.VMEM((_CHUNK,), jnp.int32),
            pltpu.VMEM((_CHUNK, _D), jnp.float32),
            pltpu.VMEM((_CHUNK, _D), jnp.float32),
            pltpu.VMEM((_CHUNK, _D), jnp.float32),
            pltpu.VMEM((_CHUNK, _D), jnp.float32),
            pltpu.VMEM_SHARED((_NPAD, _D), jnp.float32),
            pltpu.SemaphoreType.DMA,
            pltpu.SemaphoreType.DMA,
            pltpu.SemaphoreType.DMA,
            pltpu.SemaphoreType.DMA,
            pltpu.SemaphoreType.DMA,
            pltpu.SemaphoreType.DMA,
            pltpu.SemaphoreType.DMA,
            pltpu.SemaphoreType.DMA,
            pltpu.SemaphoreType.DMA,
            pltpu.SemaphoreType.DMA,
            pltpu.SemaphoreType.DMA,
            pltpu.SemaphoreType.DMA,
        ],
    )
    def agg(x_hbm, src_hbm, dst_hbm, out_hbm, srcA, srcB, srcC, srcD,
            dstA, dstB, dstC, dstD, rows0, rows1, rows2, rows3, acc_s,
            semg0, semg1, semg2, semg3, semsc0, semsc1, semsc2, semsc3,
            semiA, semiB, semiC, semiD):
        c = lax.axis_index("c")
        s = lax.axis_index("s")
        wid = c * _NS + s
        ebase = wid * _EPW

        def start_idx(j, sbuf, dbuf, sem):
            pltpu.async_copy(src_hbm.at[pl.ds(ebase + j * _CHUNK, _CHUNK)], sbuf, sem)
            pltpu.async_copy(dst_hbm.at[pl.ds(ebase + j * _CHUNK, _CHUNK)], dbuf, sem)

        def wait_idx(sbuf, dbuf, sem):
            pltpu.make_async_copy(src_hbm.at[pl.ds(0, _CHUNK)], sbuf, sem).wait()
            pltpu.make_async_copy(src_hbm.at[pl.ds(0, _CHUNK)], dbuf, sem).wait()

        def start_g(sbuf, buf, sem):
            pltpu.async_copy(x_hbm.at[sbuf], buf, sem)

        def wait_g(buf, sem):
            pltpu.make_async_copy(x_hbm.at[pl.ds(0, _CHUNK)], buf, sem).wait()

        def start_scat(dbuf, buf, sem):
            pltpu.async_copy(buf, acc_s.at[dbuf], sem, add=True)

        def wait_scat(buf, sem):
            pltpu.make_async_copy(buf, acc_s.at[pl.ds(0, _CHUNK)], sem).wait()

        sets = ((srcA, dstA, semiA), (srcB, dstB, semiB),
                (srcC, dstC, semiC), (srcD, dstD, semiD))
        rows = (rows0, rows1, rows2, rows3)
        semg = (semg0, semg1, semg2, semg3)
        semsc = (semsc0, semsc1, semsc2, semsc3)

        for k in range(4):
            start_idx(k, *sets[k])

        def zstore(i, carry):
            rows0[i // (_D // 16), pl.ds((i % (_D // 16)) * 16, 16)] = jnp.zeros(
                (16,), jnp.float32)
            return carry

        lax.fori_loop(0, _CHUNK * (_D // 16), zstore, None)

        def zcopy(j, carry):
            pltpu.sync_copy(rows0, acc_s.at[pl.ds(s * _RPS + j * _CHUNK, _CHUNK)])
            return carry

        lax.fori_loop(0, _RPS // _CHUNK, zcopy, None)

        wait_idx(*sets[0])
        start_g(srcA, rows0, semg0)
        plsc.subcore_barrier()

        for j in (0, 1):
            p, pn = j % 4, (j + 1) % 4
            wait_idx(*sets[pn])
            start_g(sets[pn][0], rows[pn], semg[pn])
            wait_g(rows[p], semg[p])
            start_scat(sets[p][1], rows[p], semsc[p])

        def quad(i, carry):
            j0 = 4 * i + 2
            for k in range(4):
                p = (2 + k) % 4
                pn = (3 + k) % 4
                pm = k
                wait_idx(*sets[pn])
                start_g(sets[pn][0], rows[pn], semg[pn])
                wait_g(rows[p], semg[p])
                start_scat(sets[p][1], rows[p], semsc[p])
                wait_scat(rows[pm], semsc[pm])
                start_idx(j0 + k + 2, sets[pm][0], sets[pm][1], sets[pm][2])
            return carry

        lax.fori_loop(0, _QUADS, quad, None)

        j = 122
        p, pn, pm = j % 4, (j + 1) % 4, (j + 2) % 4
        wait_idx(*sets[pn])
        start_g(sets[pn][0], rows[pn], semg[pn])
        wait_g(rows[p], semg[p])
        start_scat(sets[p][1], rows[p], semsc[p])
        wait_scat(rows[pm], semsc[pm])
        start_idx(124, sets[pm][0], sets[pm][1], sets[pm][2])
        j = 123
        p, pn, pm = j % 4, (j + 1) % 4, (j + 2) % 4
        wait_idx(*sets[pn])
        start_g(sets[pn][0], rows[pn], semg[pn])
        wait_g(rows[p], semg[p])
        start_scat(sets[p][1], rows[p], semsc[p])
        wait_scat(rows[pm], semsc[pm])
        wait_g(rows[0], semg[0])
        start_scat(sets[0][1], rows[0], semsc[0])
        wait_scat(rows[2], semsc[2])
        wait_scat(rows[3], semsc[3])
        wait_scat(rows[0], semsc[0])
        plsc.subcore_barrier()

        pltpu.sync_copy(
            acc_s.at[pl.ds(s * _RPS, _RPS)],
            out_hbm.at[c, pl.ds(s * _RPS, _RPS)],
        )

    return agg(x, src, dst)


def _tc_combine(partials, W, b2):
    bn = 2000
    grid = (_N // bn,)

    def body(p0_ref, p1_ref, w_ref, b_ref, o_ref):
        a = p0_ref[0] + p1_ref[0]
        h = lax.dot_general(a, w_ref[...], (((1,), (1,)), ((), ())),
                            preferred_element_type=jnp.float32)
        o_ref[...] = h + b_ref[...]

    return pl.pallas_call(
        body,
        grid=grid,
        in_specs=[
            pl.BlockSpec((1, bn, _D), lambda i: (0, i, 0)),
            pl.BlockSpec((1, bn, _D), lambda i: (1, i, 0)),
            pl.BlockSpec((_D, _D), lambda i: (0, 0)),
            pl.BlockSpec((1, _D), lambda i: (0, 0)),
        ],
        out_specs=pl.BlockSpec((bn, _D), lambda i: (i, 0)),
        out_shape=jax.ShapeDtypeStruct((_N, _D), jnp.float32),
    )(partials, partials, W, b2)


@jax.jit
def kernel(x, edge_index, W, b):
    src = edge_index[0]
    dst = edge_index[1]
    partials = _sc_aggregate(x, src, dst)
    out = _tc_combine(partials, W, b.reshape(1, _D))
    return (out,)


# speedup vs baseline: 3.8421x; 1.0100x over previous
import functools

import jax
import jax.numpy as jnp
from jax import lax
from jax.experimental import pallas as pl
from jax.experimental.pallas import tpu as pltpu
from jax.experimental.pallas import tpu_sc as plsc

_N = 10000
_E = 320000
_D = 128

_NC = 2
_NS = 16
_NW = _NC * _NS
_EPW = _E // _NW
_CHUNK = 80
_NCHUNK = _EPW // _CHUNK
_QUADS = (_NCHUNK - 5) // 4
_NPAD = 10240
_RPS = _NPAD // _NS


def _sc_aggregate(x, src, dst):
    mesh = plsc.VectorSubcoreMesh(core_axis_name="c", subcore_axis_name="s")

    @functools.partial(
        pl.kernel,
        mesh=mesh,
        out_type=jax.ShapeDtypeStruct((2, _NPAD, _D), jnp.float32),
        scratch_types=[
            pltpu.VMEM((_CHUNK,), jnp.int32),
            pltpu.VMEM((_CHUNK,), jnp.int32),
            pltpu.VMEM((_CHUNK,), jnp.int32),
            pltpu.VMEM((_CHUNK,), jnp.int32),
            pltpu.VMEM((_CHUNK,), jnp.int32),
            pltpu.VMEM((_CHUNK,), jnp.int32),
            pltpu.VMEM((_CHUNK,), jnp.int32),
            pltpu.VMEM((_CHUNK,), jnp.int32),
            pltpu.VMEM((_CHUNK, _D), jnp.float32),
            pltpu.VMEM((_CHUNK, _D), jnp.float32),
            pltpu.VMEM((_CHUNK, _D), jnp.float32),
            pltpu.VMEM((_CHUNK, _D), jnp.float32),
            pltpu.VMEM_SHARED((_NPAD, _D), jnp.float32),
            pltpu.SemaphoreType.DMA,
            pltpu.SemaphoreType.DMA,
            pltpu.SemaphoreType.DMA,
            pltpu.SemaphoreType.DMA,
            pltpu.SemaphoreType.DMA,
            pltpu.SemaphoreType.DMA,
            pltpu.SemaphoreType.DMA,
            pltpu.SemaphoreType.DMA,
            pltpu.SemaphoreType.DMA,
            pltpu.SemaphoreType.DMA,
            pltpu.SemaphoreType.DMA,
            pltpu.SemaphoreType.DMA,
        ],
    )
    def agg(x_hbm, src_hbm, dst_hbm, out_hbm, srcA, srcB, srcC, srcD,
            dstA, dstB, dstC, dstD, rows0, rows1, rows2, rows3, acc_s,
            semg0, semg1, semg2, semg3, semsc0, semsc1, semsc2, semsc3,
            semiA, semiB, semiC, semiD):
        c = lax.axis_index("c")
        s = lax.axis_index("s")
        wid = c * _NS + s
        ebase = wid * _EPW

        def start_idx(j, sbuf, dbuf, sem):
            pltpu.async_copy(src_hbm.at[pl.ds(ebase + j * _CHUNK, _CHUNK)], sbuf, sem)
            pltpu.async_copy(dst_hbm.at[pl.ds(ebase + j * _CHUNK, _CHUNK)], dbuf, sem)

        def wait_idx(sbuf, dbuf, sem):
            pltpu.make_async_copy(src_hbm.at[pl.ds(0, _CHUNK)], sbuf, sem).wait()
            pltpu.make_async_copy(src_hbm.at[pl.ds(0, _CHUNK)], dbuf, sem).wait()

        def start_g(sbuf, buf, sem):
            pltpu.async_copy(x_hbm.at[sbuf], buf, sem)

        def wait_g(buf, sem):
            pltpu.make_async_copy(x_hbm.at[pl.ds(0, _CHUNK)], buf, sem).wait()

        def start_scat(dbuf, buf, sem):
            pltpu.async_copy(buf, acc_s.at[dbuf], sem, add=True)

        def wait_scat(buf, sem):
            pltpu.make_async_copy(buf, acc_s.at[pl.ds(0, _CHUNK)], sem).wait()

        sets = ((srcA, dstA, semiA), (srcB, dstB, semiB),
                (srcC, dstC, semiC), (srcD, dstD, semiD))
        rows = (rows0, rows1, rows2, rows3)
        semg = (semg0, semg1, semg2, semg3)
        semsc = (semsc0, semsc1, semsc2, semsc3)

        for k in range(4):
            start_idx(k, *sets[k])

        def zstore(i, carry):
            rows0[i // (_D // 16), pl.ds((i % (_D // 16)) * 16, 16)] = jnp.zeros(
                (16,), jnp.float32)
            return carry

        lax.fori_loop(0, _CHUNK * (_D // 16), zstore, None)

        def zcopy(j, carry):
            pltpu.sync_copy(rows0, acc_s.at[pl.ds(s * _RPS + j * _CHUNK, _CHUNK)])
            return carry

        lax.fori_loop(0, _RPS // _CHUNK, zcopy, None)

        wait_idx(*sets[0])
        start_g(srcA, rows0, semg0)
        plsc.subcore_barrier()

        for j in (0, 1):
            p, pn = j % 4, (j + 1) % 4
            wait_idx(*sets[pn])
            start_g(sets[pn][0], rows[pn], semg[pn])
            wait_g(rows[p], semg[p])
            start_scat(sets[p][1], rows[p], semsc[p])

        def quad(i, carry):
            j0 = 4 * i + 2
            for k in range(4):
                p = (2 + k) % 4
                pn = (3 + k) % 4
                pm = k
                wait_idx(*sets[pn])
                start_g(sets[pn][0], rows[pn], semg[pn])
                wait_g(rows[p], semg[p])
                start_scat(sets[p][1], rows[p], semsc[p])
                wait_scat(rows[pm], semsc[pm])
                start_idx(j0 + k + 2, sets[pm][0], sets[pm][1], sets[pm][2])
            return carry

        lax.fori_loop(0, _QUADS, quad, None)

        j = 122
        p, pn, pm = j % 4, (j + 1) % 4, (j + 2) % 4
        wait_idx(*sets[pn])
        start_g(sets[pn][0], rows[pn], semg[pn])
        wait_g(rows[p], semg[p])
        start_scat(sets[p][1], rows[p], semsc[p])
        wait_scat(rows[pm], semsc[pm])
        start_idx(124, sets[pm][0], sets[pm][1], sets[pm][2])
        j = 123
        p, pn, pm = j % 4, (j + 1) % 4, (j + 2) % 4
        wait_idx(*sets[pn])
        start_g(sets[pn][0], rows[pn], semg[pn])
        wait_g(rows[p], semg[p])
        start_scat(sets[p][1], rows[p], semsc[p])
        wait_scat(rows[pm], semsc[pm])
        wait_g(rows[0], semg[0])
        start_scat(sets[0][1], rows[0], semsc[0])
        wait_scat(rows[2], semsc[2])
        wait_scat(rows[3], semsc[3])
        wait_scat(rows[0], semsc[0])
        plsc.subcore_barrier()

        pltpu.sync_copy(
            acc_s.at[pl.ds(s * _RPS, _RPS)],
            out_hbm.at[c, pl.ds(s * _RPS, _RPS)],
        )

    return agg(x, src, dst)


def _tc_combine(partials, W, b2):
    bn = 10000
    grid = (_N // bn,)

    def body(p0_ref, p1_ref, w_ref, b_ref, o_ref):
        a = p0_ref[0] + p1_ref[0]
        h = lax.dot_general(a, w_ref[...], (((1,), (1,)), ((), ())),
                            preferred_element_type=jnp.float32)
        o_ref[...] = h + b_ref[...]

    return pl.pallas_call(
        body,
        grid=grid,
        in_specs=[
            pl.BlockSpec((1, bn, _D), lambda i: (0, i, 0)),
            pl.BlockSpec((1, bn, _D), lambda i: (1, i, 0)),
            pl.BlockSpec((_D, _D), lambda i: (0, 0)),
            pl.BlockSpec((1, _D), lambda i: (0, 0)),
        ],
        out_specs=pl.BlockSpec((bn, _D), lambda i: (i, 0)),
        out_shape=jax.ShapeDtypeStruct((_N, _D), jnp.float32),
    )(partials, partials, W, b2)


@jax.jit
def kernel(x, edge_index, W, b):
    src = edge_index[0]
    dst = edge_index[1]
    partials = _sc_aggregate(x, src, dst)
    out = _tc_combine(partials, W, b.reshape(1, _D))
    return (out,)
